# baseline (speedup 1.0000x reference)
import functools

import jax
import jax.numpy as jnp
from jax import lax
from jax.experimental import pallas as pl
from jax.experimental.pallas import tpu as pltpu
from jax.experimental.pallas import tpu_sc as plsc

BATCH = 16384
EMBED_DIM = 32
NUM_CORES = 2
NUM_SUBCORES = 16
NUM_WORKERS = NUM_CORES * NUM_SUBCORES
B_PER_W = BATCH // NUM_WORKERS
CHUNK = 128
NCHUNK = B_PER_W // CHUNK


def _body(ri, li, ni, w_road, w_level, w_lane, out,
          idx_r, idx_l, idx_n, rows_r, rows_l, rows_n, sem):
  wid = lax.axis_index("s") * NUM_CORES + lax.axis_index("c")
  base = wid * B_PER_W

  pltpu.sync_copy(ri.at[pl.ds(wid * NCHUNK, NCHUNK)], idx_r)
  pltpu.sync_copy(li.at[pl.ds(wid * NCHUNK, NCHUNK)], idx_l)
  pltpu.sync_copy(ni.at[pl.ds(wid * NCHUNK, NCHUNK)], idx_n)

  copies = []
  for j in range(NCHUNK):
    dst = pl.ds(j * CHUNK, CHUNK)
    copies.append(pltpu.async_copy(w_road.at[idx_r.at[j]], rows_r.at[dst], sem))
    copies.append(pltpu.async_copy(w_level.at[idx_l.at[j]], rows_l.at[dst], sem))
    copies.append(pltpu.async_copy(w_lane.at[idx_n.at[j]], rows_n.at[dst], sem))
  for cp in copies:
    cp.wait()

  rows = pl.ds(base, B_PER_W)
  pltpu.sync_copy(rows_r, out.at[rows, pl.ds(0, EMBED_DIM)])
  pltpu.sync_copy(rows_l, out.at[rows, pl.ds(EMBED_DIM, EMBED_DIM)])
  pltpu.sync_copy(rows_n, out.at[rows, pl.ds(2 * EMBED_DIM, EMBED_DIM)])


@jax.jit
def _encode(road_idx, road_level, lane_num, w_road, w_level, w_lane):
  k = pl.kernel(
      _body,
      out_type=jax.ShapeDtypeStruct((BATCH, 3 * EMBED_DIM), jnp.float32),
      mesh=plsc.VectorSubcoreMesh(core_axis_name="c", subcore_axis_name="s",
                                  num_cores=NUM_CORES,
                                  num_subcores=NUM_SUBCORES),
      scratch_types=[
          pltpu.VMEM((NCHUNK, CHUNK), jnp.int32),
          pltpu.VMEM((NCHUNK, CHUNK), jnp.int32),
          pltpu.VMEM((NCHUNK, CHUNK), jnp.int32),
          pltpu.VMEM((B_PER_W, EMBED_DIM), jnp.float32),
          pltpu.VMEM((B_PER_W, EMBED_DIM), jnp.float32),
          pltpu.VMEM((B_PER_W, EMBED_DIM), jnp.float32),
          pltpu.SemaphoreType.DMA,
      ],
      compiler_params=pltpu.CompilerParams(use_tc_tiling_on_sc=False),
  )
  shape2 = (NUM_WORKERS * NCHUNK, CHUNK)
  return k(road_idx.reshape(shape2), road_level.reshape(shape2),
           lane_num.reshape(shape2), w_road, w_level, w_lane)


def kernel(road_idx, road_level, lane_num, W_road, W_level, W_lane):
  return _encode(road_idx, road_level, lane_num, W_road, W_level, W_lane)


# speedup vs baseline: 3.7729x; 3.7729x over previous
import functools

import jax
import jax.numpy as jnp
from jax import lax
from jax.experimental import pallas as pl
from jax.experimental.pallas import tpu as pltpu
from jax.experimental.pallas import tpu_sc as plsc

BATCH = 16384
EMBED_DIM = 32
NUM_CORES = 2
NUM_SUBCORES = 16
NUM_WORKERS = NUM_CORES * NUM_SUBCORES
B_PER_W = BATCH // NUM_WORKERS
CHUNK = 128
LANES = 16
NGROUP = B_PER_W // LANES


def _body(ri, li, ni, tab_t, w_level, w_lane, out,
          idx_r, idx_l, idx_n, lvl_v, lan_v, stg, blks, sems, sem0):
  wid = lax.axis_index("s") * NUM_CORES + lax.axis_index("c")
  base = wid * B_PER_W

  pltpu.sync_copy(ri.at[pl.ds(base, B_PER_W)], idx_r)
  pltpu.sync_copy(li.at[pl.ds(base, B_PER_W)], idx_l)
  pltpu.sync_copy(ni.at[pl.ds(base, B_PER_W)], idx_n)
  pltpu.sync_copy(w_level, lvl_v)
  pltpu.sync_copy(w_lane, lan_v)

  f_lo = lax.iota(jnp.int32, LANES)
  f_hi = f_lo + LANES

  def group(g, _):
    off = pl.multiple_of(g * LANES, LANES)
    rv = idx_r[pl.ds(off, LANES)]
    lv = idx_l[pl.ds(off, LANES)]
    nv = idx_n[pl.ds(off, LANES)]
    cv = rv & (CHUNK - 1)
    tv = rv >> 7
    for s in range(LANES):
      col0 = pl.multiple_of(tv[s] * CHUNK, CHUNK)
      pltpu.async_copy(tab_t.at[:, pl.ds(col0, CHUNK)],
                       blks.at[s], sems.at[s])
    for s in range(LANES):
      pltpu.make_async_copy(tab_t.at[:, pl.ds(0, CHUNK)],
                            blks.at[s], sems.at[s]).wait()
      blk = blks.at[s]
      c = jnp.full((LANES,), cv[s], jnp.int32)
      l = jnp.full((LANES,), lv[s], jnp.int32)
      n = jnp.full((LANES,), nv[s], jnp.int32)
      col = jnp.full((LANES,), off + s, jnp.int32)
      plsc.store_scatter(stg, [f_lo, col], plsc.load_gather(blk, [f_lo, c]))
      plsc.store_scatter(stg, [f_hi, col], plsc.load_gather(blk, [f_hi, c]))
      plsc.store_scatter(stg, [f_lo + 32, col],
                         plsc.load_gather(lvl_v, [l, f_lo]))
      plsc.store_scatter(stg, [f_hi + 32, col],
                         plsc.load_gather(lvl_v, [l, f_hi]))
      plsc.store_scatter(stg, [f_lo + 64, col],
                         plsc.load_gather(lan_v, [n, f_lo]))
      plsc.store_scatter(stg, [f_hi + 64, col],
                         plsc.load_gather(lan_v, [n, f_hi]))
    return _

  lax.fori_loop(0, NGROUP, group, 0)

  pltpu.async_copy(stg, out.at[:, pl.ds(base, B_PER_W)], sem0).wait()


@jax.jit
def _encode(road_idx, road_level, lane_num, w_road, w_level, w_lane):
  k = pl.kernel(
      _body,
      out_type=jax.ShapeDtypeStruct((3 * EMBED_DIM, BATCH), jnp.float32),
      mesh=plsc.VectorSubcoreMesh(core_axis_name="c", subcore_axis_name="s",
                                  num_cores=NUM_CORES,
                                  num_subcores=NUM_SUBCORES),
      scratch_types=[
          pltpu.VMEM((B_PER_W,), jnp.int32),
          pltpu.VMEM((B_PER_W,), jnp.int32),
          pltpu.VMEM((B_PER_W,), jnp.int32),
          pltpu.VMEM((10, EMBED_DIM), jnp.float32),
          pltpu.VMEM((16, EMBED_DIM), jnp.float32),
          pltpu.VMEM((3 * EMBED_DIM, B_PER_W), jnp.float32),
          pltpu.VMEM((LANES, EMBED_DIM, CHUNK), jnp.float32),
          pltpu.SemaphoreType.DMA((LANES,)),
          pltpu.SemaphoreType.DMA,
      ],
      compiler_params=pltpu.CompilerParams(needs_layout_passes=False),
  )
  out_t = k(road_idx, road_level, lane_num, w_road.T, w_level, w_lane)
  return out_t.T


def kernel(road_idx, road_level, lane_num, W_road, W_level, W_lane):
  return _encode(road_idx, road_level, lane_num, W_road, W_level, W_lane)


# speedup vs baseline: 4.4770x; 1.1866x over previous
import functools

import jax
import jax.numpy as jnp
from jax import lax
from jax.experimental import pallas as pl
from jax.experimental.pallas import tpu as pltpu
from jax.experimental.pallas import tpu_sc as plsc

BATCH = 16384
EMBED_DIM = 32
NUM_CORES = 2
NUM_SUBCORES = 16
NUM_WORKERS = NUM_CORES * NUM_SUBCORES
B_PER_W = BATCH // NUM_WORKERS
CHUNK = 128
CHUNK_SHIFT = 7
LANES = 16
NGROUP = B_PER_W // LANES


def _body(ri, li, ni, tab_t, w_level, w_lane, out,
          idx_r, idx_l, idx_n, lvl_v, lan_v, stg, blks, sems, sem0):
  wid = lax.axis_index("s") * NUM_CORES + lax.axis_index("c")
  base = wid * B_PER_W

  pltpu.sync_copy(ri.at[pl.ds(base, B_PER_W)], idx_r)
  pltpu.sync_copy(li.at[pl.ds(base, B_PER_W)], idx_l)
  pltpu.sync_copy(ni.at[pl.ds(base, B_PER_W)], idx_n)
  pltpu.sync_copy(w_level, lvl_v)
  pltpu.sync_copy(w_lane, lan_v)

  f_lo = lax.iota(jnp.int32, LANES)
  f_hi = f_lo + LANES

  def issue(s, tv):
    col0 = pl.multiple_of(tv[s] * CHUNK, CHUNK)
    pltpu.async_copy(tab_t.at[:, pl.ds(col0, CHUNK)],
                     blks.at[s], sems.at[s])

  def lvl_lane(g):
    off = pl.multiple_of(g * LANES, LANES)
    lv = idx_l[pl.ds(off, LANES)]
    nv = idx_n[pl.ds(off, LANES)]
    for f in range(EMBED_DIM):
      fb = jnp.full((LANES,), f, jnp.int32)
      stg[32 + f, pl.ds(off, LANES)] = plsc.load_gather(lvl_v, [lv, fb])
      stg[64 + f, pl.ds(off, LANES)] = plsc.load_gather(lan_v, [nv, fb])

  def extract(g, s, cv):
    e = jnp.full((LANES,), g * LANES + s, jnp.int32)
    sb = jnp.full((LANES,), s, jnp.int32)
    c = jnp.full((LANES,), cv[s], jnp.int32)
    pltpu.make_async_copy(tab_t.at[:, pl.ds(0, CHUNK)],
                          blks.at[s], sems.at[s]).wait()
    plsc.store_scatter(stg, [f_lo, e],
                       plsc.load_gather(blks, [sb, f_lo, c]))
    plsc.store_scatter(stg, [f_hi, e],
                       plsc.load_gather(blks, [sb, f_hi, c]))

  tv0 = idx_r[pl.ds(0, LANES)] >> CHUNK_SHIFT
  for s in range(LANES):
    issue(s, tv0)

  def group(g, _):
    off = pl.multiple_of(g * LANES, LANES)
    rv = idx_r[pl.ds(off, LANES)]
    cv = rv & (CHUNK - 1)
    off1 = pl.multiple_of(off + LANES, LANES)
    tv1 = idx_r[pl.ds(off1, LANES)] >> CHUNK_SHIFT
    lvl_lane(g)
    for s in range(LANES):
      extract(g, s, cv)
      issue(s, tv1)
    return _

  lax.fori_loop(0, NGROUP - 1, group, 0)

  g_last = NGROUP - 1
  rv = idx_r[pl.ds(g_last * LANES, LANES)]
  cv = rv & (CHUNK - 1)
  lvl_lane(g_last)
  for s in range(LANES):
    extract(g_last, s, cv)

  pltpu.async_copy(stg, out.at[:, pl.ds(base, B_PER_W)], sem0).wait()


@jax.jit
def _encode(road_idx, road_level, lane_num, w_road, w_level, w_lane):
  k = pl.kernel(
      _body,
      out_type=jax.ShapeDtypeStruct((3 * EMBED_DIM, BATCH), jnp.float32),
      mesh=plsc.VectorSubcoreMesh(core_axis_name="c", subcore_axis_name="s",
                                  num_cores=NUM_CORES,
                                  num_subcores=NUM_SUBCORES),
      scratch_types=[
          pltpu.VMEM((B_PER_W,), jnp.int32),
          pltpu.VMEM((B_PER_W,), jnp.int32),
          pltpu.VMEM((B_PER_W,), jnp.int32),
          pltpu.VMEM((10, EMBED_DIM), jnp.float32),
          pltpu.VMEM((16, EMBED_DIM), jnp.float32),
          pltpu.VMEM((3 * EMBED_DIM, B_PER_W), jnp.float32),
          pltpu.VMEM((LANES, EMBED_DIM, CHUNK), jnp.float32),
          pltpu.SemaphoreType.DMA((LANES,)),
          pltpu.SemaphoreType.DMA,
      ],
      compiler_params=pltpu.CompilerParams(needs_layout_passes=False),
  )
  out_t = k(road_idx, road_level, lane_num, w_road.T, w_level, w_lane)
  return out_t.T


def kernel(road_idx, road_level, lane_num, W_road, W_level, W_lane):
  return _encode(road_idx, road_level, lane_num, W_road, W_level, W_lane)


# speedup vs baseline: 4.5212x; 1.0099x over previous
import functools

import jax
import jax.numpy as jnp
from jax import lax
from jax.experimental import pallas as pl
from jax.experimental.pallas import tpu as pltpu
from jax.experimental.pallas import tpu_sc as plsc

BATCH = 16384
EMBED_DIM = 32
NUM_CORES = 2
NUM_SUBCORES = 16
NUM_WORKERS = NUM_CORES * NUM_SUBCORES
B_PER_W = BATCH // NUM_WORKERS
CHUNK = 128
CHUNK_SHIFT = 7
LANES = 16
NGROUP = B_PER_W // LANES


def _body(ri, li, ni, tab_t, w_level, w_lane, out,
          idx_r, idx_l, idx_n, lvl_v, lan_v, stg, blks, sems, sem0):
  wid = lax.axis_index("s") * NUM_CORES + lax.axis_index("c")
  base = wid * B_PER_W

  pltpu.sync_copy(ri.at[pl.ds(base, B_PER_W)], idx_r)
  pltpu.sync_copy(li.at[pl.ds(base, B_PER_W)], idx_l)
  pltpu.sync_copy(ni.at[pl.ds(base, B_PER_W)], idx_n)
  pltpu.sync_copy(w_level, lvl_v)
  pltpu.sync_copy(w_lane, lan_v)

  f_lo = lax.iota(jnp.int32, LANES)
  f_hi = f_lo + LANES

  def issue(s, tv):
    col0 = pl.multiple_of(tv[s] * CHUNK, CHUNK)
    pltpu.async_copy(tab_t.at[:, pl.ds(col0, CHUNK)],
                     blks.at[s], sems.at[s])

  def lvl_lane(g, lv, nv, s):
    off = pl.multiple_of(g * LANES, LANES)
    for f in (2 * s, 2 * s + 1):
      fb = jnp.full((LANES,), f, jnp.int32)
      stg[32 + f, pl.ds(off, LANES)] = plsc.load_gather(lvl_v, [lv, fb])
      stg[64 + f, pl.ds(off, LANES)] = plsc.load_gather(lan_v, [nv, fb])

  def extract(g, s, cv):
    e = jnp.full((LANES,), g * LANES + s, jnp.int32)
    sb = jnp.full((LANES,), s, jnp.int32)
    c = jnp.full((LANES,), cv[s], jnp.int32)
    pltpu.make_async_copy(tab_t.at[:, pl.ds(0, CHUNK)],
                          blks.at[s], sems.at[s]).wait()
    plsc.store_scatter(stg, [f_lo, e],
                       plsc.load_gather(blks, [sb, f_lo, c]))
    plsc.store_scatter(stg, [f_hi, e],
                       plsc.load_gather(blks, [sb, f_hi, c]))

  tv0 = idx_r[pl.ds(0, LANES)] >> CHUNK_SHIFT
  for s in range(LANES):
    issue(s, tv0)

  def group(g, _):
    off = pl.multiple_of(g * LANES, LANES)
    rv = idx_r[pl.ds(off, LANES)]
    lv = idx_l[pl.ds(off, LANES)]
    nv = idx_n[pl.ds(off, LANES)]
    cv = rv & (CHUNK - 1)
    off1 = pl.multiple_of(off + LANES, LANES)
    tv1 = idx_r[pl.ds(off1, LANES)] >> CHUNK_SHIFT
    for s in range(LANES):
      extract(g, s, cv)
      issue(s, tv1)
      lvl_lane(g, lv, nv, s)
    return _

  lax.fori_loop(0, NGROUP - 1, group, 0)

  g_last = NGROUP - 1
  rv = idx_r[pl.ds(g_last * LANES, LANES)]
  lv = idx_l[pl.ds(g_last * LANES, LANES)]
  nv = idx_n[pl.ds(g_last * LANES, LANES)]
  cv = rv & (CHUNK - 1)
  for s in range(LANES):
    extract(g_last, s, cv)
    lvl_lane(g_last, lv, nv, s)

  pltpu.async_copy(stg, out.at[:, pl.ds(base, B_PER_W)], sem0).wait()


@jax.jit
def _encode(road_idx, road_level, lane_num, w_road, w_level, w_lane):
  k = pl.kernel(
      _body,
      out_type=jax.ShapeDtypeStruct((3 * EMBED_DIM, BATCH), jnp.float32),
      mesh=plsc.VectorSubcoreMesh(core_axis_name="c", subcore_axis_name="s",
                                  num_cores=NUM_CORES,
                                  num_subcores=NUM_SUBCORES),
      scratch_types=[
          pltpu.VMEM((B_PER_W,), jnp.int32),
          pltpu.VMEM((B_PER_W,), jnp.int32),
          pltpu.VMEM((B_PER_W,), jnp.int32),
          pltpu.VMEM((10, EMBED_DIM), jnp.float32),
          pltpu.VMEM((16, EMBED_DIM), jnp.float32),
          pltpu.VMEM((3 * EMBED_DIM, B_PER_W), jnp.float32),
          pltpu.VMEM((LANES, EMBED_DIM, CHUNK), jnp.float32),
          pltpu.SemaphoreType.DMA((LANES,)),
          pltpu.SemaphoreType.DMA,
      ],
      compiler_params=pltpu.CompilerParams(needs_layout_passes=False),
  )
  out_t = k(road_idx, road_level, lane_num, w_road.T, w_level, w_lane)
  return out_t.T


def kernel(road_idx, road_level, lane_num, W_road, W_level, W_lane):
  return _encode(road_idx, road_level, lane_num, W_road, W_level, W_lane)
